# baseline (speedup 1.0000x reference)
import functools

import jax
import jax.numpy as jnp
from jax import lax
from jax.experimental import pallas as pl
from jax.experimental.pallas import tpu as pltpu
from jax.experimental.pallas import tpu_sc as plsc

NUM_CORES = 2
NUM_SUBCORES = 16
NUM_TILES = NUM_CORES * NUM_SUBCORES
CHUNK = 64


def _sc_gather_rows(table, flat_idx, n_rows, h):
    rows_per_tile = n_rows // NUM_TILES
    chunks = rows_per_tile // CHUNK
    mesh = plsc.VectorSubcoreMesh(core_axis_name="c", subcore_axis_name="s")

    @functools.partial(
        pl.kernel,
        out_type=jax.ShapeDtypeStruct((n_rows, h), table.dtype),
        mesh=mesh,
        scratch_types=[
            pltpu.VMEM((rows_per_tile,), jnp.int32),
            pltpu.VMEM((2, CHUNK, h), jnp.float32),
            pltpu.SemaphoreType.DMA,
            pltpu.SemaphoreType.DMA,
            pltpu.SemaphoreType.DMA,
            pltpu.SemaphoreType.DMA,
        ],
    )
    def k(table_hbm, idx_hbm, out_hbm, idx_v, rows_v, sg0, sg1, sw0, sw1):
        wid = lax.axis_index("s") * NUM_CORES + lax.axis_index("c")
        base = wid * rows_per_tile
        pltpu.sync_copy(idx_hbm.at[pl.ds(base, rows_per_tile)], idx_v)

        sems_g = (sg0, sg1)
        sems_w = (sw0, sw1)
        h_gather = [None, None]
        h_write = [None, None]
        for g in range(chunks + 1):
            b = g & 1
            if g < chunks:
                if h_write[b] is not None:
                    h_write[b].wait()
                h_gather[b] = pltpu.async_copy(
                    table_hbm.at[idx_v.at[pl.ds(g * CHUNK, CHUNK)]],
                    rows_v.at[b],
                    sems_g[b],
                )
            if g >= 1:
                pb = (g - 1) & 1
                h_gather[pb].wait()
                h_write[pb] = pltpu.async_copy(
                    rows_v.at[pb],
                    out_hbm.at[pl.ds(base + (g - 1) * CHUNK, CHUNK)],
                    sems_w[pb],
                )
        for b in range(2):
            if h_write[b] is not None:
                h_write[b].wait()

    return k(table, flat_idx)


def kernel(inputs, mask_indices, un_masked_indices, mst):
    b, n_vis, h = inputs.shape
    m = mask_indices.shape[0]
    n_total = m + n_vis

    idx = jnp.concatenate([mask_indices, un_masked_indices]).astype(jnp.int32)
    table = jnp.concatenate(
        [inputs.reshape(b * n_vis, h), mst.reshape(1, h).astype(inputs.dtype)], axis=0
    )
    is_vis = idx >= m
    local = jnp.where(is_vis, idx - m, 0)
    flat = local[None, :] + jnp.arange(b, dtype=jnp.int32)[:, None] * n_vis
    flat = jnp.where(is_vis[None, :], flat, b * n_vis).reshape(-1)

    out = _sc_gather_rows(table, flat, b * n_total, h)
    return out.reshape(b, n_total, h)


# speedup vs baseline: 8.6109x; 8.6109x over previous
import functools

import jax
import jax.numpy as jnp
from jax import lax
from jax.experimental import pallas as pl
from jax.experimental.pallas import tpu as pltpu
from jax.experimental.pallas import tpu_sc as plsc

NUM_CORES = 2
NUM_SUBCORES = 16
NUM_TILES = NUM_CORES * NUM_SUBCORES
CHUNK = 64


def _sc_gather_rows(table, flat_idx, n_rows, h):
    rows_per_tile = n_rows // NUM_TILES
    chunks = rows_per_tile // CHUNK
    mesh = plsc.VectorSubcoreMesh(core_axis_name="c", subcore_axis_name="s")

    @functools.partial(
        pl.kernel,
        out_type=jax.ShapeDtypeStruct((n_rows, h), table.dtype),
        mesh=mesh,
        scratch_types=[
            pltpu.VMEM((rows_per_tile,), jnp.int32),
            pltpu.VMEM((2, CHUNK, h), jnp.float32),
            pltpu.SemaphoreType.DMA,
            pltpu.SemaphoreType.DMA,
            pltpu.SemaphoreType.DMA,
            pltpu.SemaphoreType.DMA,
        ],
    )
    def k(table_hbm, idx_hbm, out_hbm, idx_v, rows_v, sg0, sg1, sw0, sw1):
        wid = lax.axis_index("s") * NUM_CORES + lax.axis_index("c")
        base = wid * rows_per_tile
        pltpu.sync_copy(idx_hbm.at[pl.ds(base, rows_per_tile)], idx_v)

        sems_g = (sg0, sg1)
        sems_w = (sw0, sw1)
        h_gather = [None, None]
        h_write = [None, None]
        for g in range(chunks + 1):
            b = g & 1
            if g < chunks:
                if h_write[b] is not None:
                    h_write[b].wait()
                h_gather[b] = pltpu.async_copy(
                    table_hbm.at[idx_v.at[pl.ds(g * CHUNK, CHUNK)]],
                    rows_v.at[b],
                    sems_g[b],
                )
            if g >= 1:
                pb = (g - 1) & 1
                h_gather[pb].wait()
                h_write[pb] = pltpu.async_copy(
                    rows_v.at[pb],
                    out_hbm.at[pl.ds(base + (g - 1) * CHUNK, CHUNK)],
                    sems_w[pb],
                )
        for b in range(2):
            if h_write[b] is not None:
                h_write[b].wait()

    return k(table, flat_idx)


def kernel(inputs, mask_indices, un_masked_indices, mst):
    b, n_vis, h = inputs.shape
    m = mask_indices.shape[0]
    n_total = m + n_vis

    idx = jnp.concatenate([mask_indices, un_masked_indices]).astype(jnp.int32)
    reps = 256
    table = jnp.concatenate(
        [
            inputs.reshape(b * n_vis, h),
            jnp.broadcast_to(mst.reshape(1, h).astype(inputs.dtype), (reps, h)),
        ],
        axis=0,
    )
    is_vis = idx >= m
    local = jnp.where(is_vis, idx - m, 0)
    flat = local[None, :] + jnp.arange(b, dtype=jnp.int32)[:, None] * n_vis
    rows = jnp.arange(b * n_total, dtype=jnp.int32).reshape(b, n_total)
    flat = jnp.where(is_vis[None, :], flat, b * n_vis + (rows % reps)).reshape(-1)

    out = _sc_gather_rows(table, flat, b * n_total, h)
    return out.reshape(b, n_total, h)


# speedup vs baseline: 12.9127x; 1.4996x over previous
import dataclasses
import functools

import jax
import jax.numpy as jnp
from jax import lax
from jax.experimental import pallas as pl
from jax.experimental.pallas import tpu as pltpu
from jax.experimental.pallas import tpu_sc as plsc

NUM_CORES = 2
NUM_SUBCORES = 16
NUM_TILES = NUM_CORES * NUM_SUBCORES
BATCHES_PER_TILE = 2
CHUNK_V = 32
CHUNK_M = 64
LANES = 16


def _sc_cmask(n_batch, n_vis, n_tok, h):
    vslots = n_tok // CHUNK_V
    mslots = n_tok // CHUNK_M
    mesh = plsc.VectorSubcoreMesh(core_axis_name="c", subcore_axis_name="s")
    cp = pltpu.CompilerParams()
    if "needs_layout_passes" in pltpu.CompilerParams.__dataclass_fields__:
        cp = dataclasses.replace(cp, needs_layout_passes=False)

    @functools.partial(
        pl.kernel,
        out_type=jax.ShapeDtypeStruct((n_batch * n_tok, h), jnp.float32),
        mesh=mesh,
        compiler_params=cp,
        scratch_types=[
            pltpu.VMEM((n_tok,), jnp.int32),
            pltpu.VMEM((n_tok,), jnp.int32),
            pltpu.VMEM((n_tok,), jnp.int32),
            pltpu.VMEM((LANES,), jnp.int32),
            pltpu.VMEM((CHUNK_M, h), jnp.float32),
            pltpu.VMEM((2, CHUNK_V, h), jnp.float32),
            pltpu.VMEM((BATCHES_PER_TILE * vslots, CHUNK_V), jnp.int32),
            pltpu.VMEM((BATCHES_PER_TILE * vslots, CHUNK_V), jnp.int32),
            pltpu.VMEM((BATCHES_PER_TILE * mslots, CHUNK_M), jnp.int32),
            pltpu.SemaphoreType.DMA,
            pltpu.SemaphoreType.DMA,
            pltpu.SemaphoreType.DMA,
        ],
    )
    def k(inp_hbm, mst_hbm, vpos_hbm, src_hbm, mpos_hbm, cnt_hbm, out_hbm,
          vpos_v, src_v, mpos_v, cnt_v, mstblk_v, rows_v,
          vsrc2d, vdst2d, mdst2d, sem_m, sem_v0, sem_v1):
        wid = lax.axis_index("s") * NUM_CORES + lax.axis_index("c")

        pltpu.sync_copy(vpos_hbm, vpos_v)
        pltpu.sync_copy(src_hbm, src_v)
        pltpu.sync_copy(mpos_hbm, mpos_v)
        pltpu.sync_copy(cnt_hbm, cnt_v)

        lane = lax.iota(jnp.int32, LANES)
        cvec = cnt_v[pl.ds(0, LANES)]
        kv = jnp.sum(jnp.where(lane == 0, cvec, 0))
        km = jnp.sum(jnp.where(lane == 1, cvec, 0))

        pltpu.sync_copy(mst_hbm, mstblk_v)

        sems_v = (sem_v0, sem_v1)
        for nb in range(BATCHES_PER_TILE):
            b = wid * BATCHES_PER_TILE + nb
            out_off = b * n_tok
            in_off = b * n_vis

            @pl.loop(0, mslots)
            def _(s, nb=nb, out_off=out_off):
                @pl.when(s * CHUNK_M < km)
                def _():
                    row = nb * mslots + s
                    for g in range(CHUNK_M // LANES):
                        pos = mpos_v[pl.ds(s * CHUNK_M + g * LANES, LANES)]
                        mdst2d[row, pl.ds(g * LANES, LANES)] = pos + out_off
                    pltpu.async_copy(mstblk_v, out_hbm.at[mdst2d.at[row]], sem_m)

            @pl.loop(0, vslots, step=2)
            def _(s0, nb=nb, out_off=out_off, in_off=in_off):
                for p in range(2):
                    s = s0 + p

                    @pl.when(jnp.logical_and(s * CHUNK_V < kv, s >= 2))
                    def _(p=p, s=s):
                        pltpu.make_async_copy(
                            rows_v.at[p], out_hbm.at[pl.ds(0, CHUNK_V)], sems_v[p]
                        ).wait()

                    @pl.when(s * CHUNK_V < kv)
                    def _(p=p, s=s):
                        row = nb * vslots + s
                        for g in range(CHUNK_V // LANES):
                            sl = pl.ds(s * CHUNK_V + g * LANES, LANES)
                            vsrc2d[row, pl.ds(g * LANES, LANES)] = src_v[sl] + in_off
                            vdst2d[row, pl.ds(g * LANES, LANES)] = vpos_v[sl] + out_off
                        pltpu.sync_copy(inp_hbm.at[vsrc2d.at[row]], rows_v.at[p])
                        pltpu.async_copy(rows_v.at[p], out_hbm.at[vdst2d.at[row]], sems_v[p])

            for p in range(2):
                @pl.when(p * CHUNK_V < kv)
                def _(p=p):
                    pltpu.make_async_copy(
                        rows_v.at[p], out_hbm.at[pl.ds(0, CHUNK_V)], sems_v[p]
                    ).wait()

        @pl.loop(0, mslots)
        def _(s):
            @pl.when(s * CHUNK_M < km)
            def _():
                for _ in range(BATCHES_PER_TILE):
                    pltpu.make_async_copy(
                        mstblk_v, out_hbm.at[pl.ds(0, CHUNK_M)], sem_m
                    ).wait()

    return k


def kernel(inputs, mask_indices, un_masked_indices, mst):
    b, n_vis, h = inputs.shape
    m = mask_indices.shape[0]
    n_tok = m + n_vis

    idx = jnp.concatenate([mask_indices, un_masked_indices]).astype(jnp.int32)
    is_mst = idx < m
    ar = jnp.arange(n_tok, dtype=jnp.int32)
    kv = jnp.sum((~is_mst).astype(jnp.int32))
    km = n_tok - kv
    perm_v = jnp.argsort(jnp.where(is_mst, 1, 0), stable=True).astype(jnp.int32)
    perm_m = jnp.argsort(jnp.where(is_mst, 0, 1), stable=True).astype(jnp.int32)
    selv = jnp.where(ar < kv, ar, ar % jnp.maximum(kv, 1))
    selm = jnp.where(ar < km, ar, ar % jnp.maximum(km, 1))
    vis_pos = perm_v[selv]
    mst_pos = perm_m[selm]
    src_loc = jnp.clip(idx[vis_pos] - m, 0, n_vis - 1)
    cnt = jnp.zeros((LANES,), jnp.int32).at[0].set(kv).at[1].set(km)

    out = _sc_cmask(b, n_vis, n_tok, h)(
        inputs.reshape(b * n_vis, h),
        jnp.broadcast_to(mst.reshape(1, h).astype(inputs.dtype), (CHUNK_M, h)),
        vis_pos, src_loc, mst_pos, cnt,
    )
    return out.reshape(b, n_tok, h)
